# baseline (speedup 1.0000x reference)
import functools

import jax
import jax.numpy as jnp
from jax import lax
from jax.experimental import pallas as pl
from jax.experimental.pallas import tpu as pltpu
from jax.experimental.pallas import tpu_sc as plsc

_B = 16384
_D = 64
_NW = 32
_BPW = _B // _NW
_CH = 128
_NCH = _BPW // _CH

_BS = 1024


def _sc_gather_body(uidx_hbm, gidx_hbm, ptab_hbm, gtab_hbm,
                    u_out, g_out,
                    uidx_v, gidx_v, urows_v, grows_v, sem_u, sem_g):
    wid = lax.axis_index("s") * 2 + lax.axis_index("c")
    base = wid * _BPW
    pltpu.sync_copy(uidx_hbm.at[wid], uidx_v)
    pltpu.sync_copy(gidx_hbm.at[wid], gidx_v)
    copies = []
    for j in range(_NCH):
        copies.append(pltpu.async_copy(
            ptab_hbm.at[uidx_v.at[j]], urows_v.at[pl.ds(j * _CH, _CH)], sem_u))
        copies.append(pltpu.async_copy(
            gtab_hbm.at[gidx_v.at[j]], grows_v.at[pl.ds(j * _CH, _CH)], sem_g))
    for c in copies:
        c.wait()
    pltpu.sync_copy(urows_v, u_out.at[pl.ds(base, _BPW)])
    pltpu.sync_copy(grows_v, g_out.at[pl.ds(base, _BPW)])


@functools.cache
def _make_sc_gather():
    return functools.partial(
        pl.kernel,
        mesh=plsc.VectorSubcoreMesh(core_axis_name="c", subcore_axis_name="s"),
        compiler_params=pltpu.CompilerParams(use_tc_tiling_on_sc=False),
        out_type=[
            jax.ShapeDtypeStruct((_B, _D), jnp.float32),
            jax.ShapeDtypeStruct((_B, _D), jnp.float32),
        ],
        scratch_types=[
            pltpu.VMEM((_NCH, _CH), jnp.int32),
            pltpu.VMEM((_NCH, _CH), jnp.int32),
            pltpu.VMEM((_BPW, _D), jnp.float32),
            pltpu.VMEM((_BPW, _D), jnp.float32),
            pltpu.SemaphoreType.DMA,
            pltpu.SemaphoreType.DMA,
        ],
    )(_sc_gather_body)


def _mlp_body(u_ref, g_ref, w1a_ref, w1b_ref, b1_ref, w2_ref, b2_ref,
              w3_ref, b3_ref, w4_ref, b4_ref, o_ref):
    f32 = jnp.float32
    h = jnp.maximum(
        jnp.dot(u_ref[...], w1a_ref[...], preferred_element_type=f32)
        + jnp.dot(g_ref[...], w1b_ref[...], preferred_element_type=f32)
        + b1_ref[...], 0.0)
    h = jnp.maximum(
        jnp.dot(h, w2_ref[...], preferred_element_type=f32) + b2_ref[...], 0.0)
    h = jnp.maximum(
        jnp.dot(h, w3_ref[...], preferred_element_type=f32) + b3_ref[...], 0.0)
    o_ref[...] = jnp.dot(h, w4_ref[...], preferred_element_type=f32) + b4_ref[...]


def _mlp(u_emb, g_emb, w1a, w1b, b1, w2, b2, w3, b3, w4, b4):
    full = lambda shape: pl.BlockSpec(shape, lambda i: (0, 0))
    return pl.pallas_call(
        _mlp_body,
        grid=(_B // _BS,),
        in_specs=[
            pl.BlockSpec((_BS, _D), lambda i: (i, 0)),
            pl.BlockSpec((_BS, _D), lambda i: (i, 0)),
            full(w1a.shape), full(w1b.shape), full(b1.shape),
            full(w2.shape), full(b2.shape),
            full(w3.shape), full(b3.shape),
            full(w4.shape), full(b4.shape),
        ],
        out_specs=pl.BlockSpec((_BS, 1), lambda i: (i, 0)),
        out_shape=jax.ShapeDtypeStruct((_B, 1), jnp.float32),
    )(u_emb, g_emb, w1a, w1b, b1, w2, b2, w3, b3, w4, b4)


def kernel(user, game, player_table, game_table, W1, b1, W2, b2, W3, b3, W4, b4):
    uidx = user.reshape(_NW, _NCH, _CH)
    gidx = game.reshape(_NW, _NCH, _CH)
    u_emb, g_emb = _make_sc_gather()(uidx, gidx, player_table, game_table)
    return _mlp(u_emb, g_emb,
                W1[:_D], W1[_D:], b1.reshape(1, -1),
                W2, b2.reshape(1, -1),
                W3, b3.reshape(1, -1),
                W4, b4.reshape(1, 1))


# speedup vs baseline: 1.7802x; 1.7802x over previous
import functools

import jax
import jax.numpy as jnp
from jax import lax
from jax.experimental import pallas as pl
from jax.experimental.pallas import tpu as pltpu
from jax.experimental.pallas import tpu_sc as plsc

_B = 16384
_D = 64
_NW = 32
_BPW = _B // _NW
_CH = 64
_NCH = _BPW // _CH
_L = 16

_BS = 1024


def _extract_chunk(k, idx_v, slabs_v, compact_v):
    for g in range(_CH // _L):
        off = k * _CH + g * _L
        idx16 = idx_v[pl.ds(off, _L)]
        s_vec = lax.bitwise_and(idx16, jnp.int32(7))
        d0 = lax.iota(jnp.int32, _L) + jnp.int32(g * _L)
        col_vec = lax.iota(jnp.int32, _L) + off
        for c in range(_D):
            val = plsc.load_gather(
                slabs_v, [d0, s_vec, jnp.full((_L,), c, jnp.int32)])
            plsc.store_scatter(
                compact_v, [jnp.full((_L,), c, jnp.int32), col_vec], val)


def _gather_table(idx_hbm, tab3_hbm, out_hbm, base, idx_v, t_v, slabs_v,
                  compact_v, sem):
    pltpu.sync_copy(idx_hbm.at[pl.ds(base, _BPW)], idx_v)
    lane = lax.iota(jnp.int32, _L)

    def body(k, carry):
        copies = []
        for g in range(_CH // _L):
            t16 = lax.shift_right_logical(
                idx_v[pl.ds(k * _CH + g * _L, _L)], jnp.int32(3))
            for l in range(_L):
                t_s = jnp.sum(jnp.where(lane == l, t16, 0))
                j = g * _L + l
                copies.append(pltpu.async_copy(
                    tab3_hbm.at[pl.ds(t_s, 1)],
                    slabs_v.at[pl.ds(j, 1)], sem))
        for c in copies:
            c.wait()
        _extract_chunk(k, idx_v, slabs_v, compact_v)
        return carry

    lax.fori_loop(0, _NCH, body, jnp.int32(0))
    pltpu.sync_copy(compact_v, out_hbm.at[:, pl.ds(base, _BPW)])


def _sc_gather_body(uidx_hbm, gidx_hbm, ptab3_hbm, gtab3_hbm,
                    ut_out, gt_out,
                    idx_v, t_v, slabs_v, compact_v, sem):
    wid = lax.axis_index("s") * 2 + lax.axis_index("c")
    base = wid * _BPW
    _gather_table(uidx_hbm, ptab3_hbm, ut_out, base,
                  idx_v, t_v, slabs_v, compact_v, sem)
    _gather_table(gidx_hbm, gtab3_hbm, gt_out, base,
                  idx_v, t_v, slabs_v, compact_v, sem)


@functools.cache
def _make_sc_gather():
    return functools.partial(
        pl.kernel,
        mesh=plsc.VectorSubcoreMesh(core_axis_name="c", subcore_axis_name="s"),
        compiler_params=pltpu.CompilerParams(needs_layout_passes=False),
        out_type=[
            jax.ShapeDtypeStruct((_D, _B), jnp.float32),
            jax.ShapeDtypeStruct((_D, _B), jnp.float32),
        ],
        scratch_types=[
            pltpu.VMEM((_BPW,), jnp.int32),
            pltpu.VMEM((_BPW,), jnp.int32),
            pltpu.VMEM((_CH, 8, _D), jnp.float32),
            pltpu.VMEM((_D, _BPW), jnp.float32),
            pltpu.SemaphoreType.DMA,
        ],
    )(_sc_gather_body)


def _mlp_body(u_ref, g_ref, w1a_ref, w1b_ref, b1_ref, w2_ref, b2_ref,
              w3_ref, b3_ref, w4_ref, b4_ref, o_ref):
    f32 = jnp.float32
    h = jnp.maximum(
        jnp.dot(w1a_ref[...], u_ref[...], preferred_element_type=f32)
        + jnp.dot(w1b_ref[...], g_ref[...], preferred_element_type=f32)
        + b1_ref[...], 0.0)
    h = jnp.maximum(
        jnp.dot(w2_ref[...], h, preferred_element_type=f32) + b2_ref[...], 0.0)
    h = jnp.maximum(
        jnp.dot(w3_ref[...], h, preferred_element_type=f32) + b3_ref[...], 0.0)
    o_ref[...] = jnp.dot(w4_ref[...], h, preferred_element_type=f32) + b4_ref[...]


def _mlp_t(ut, gt, w1at, w1bt, b1c, w2t, b2c, w3t, b3c, w4t, b4c):
    full = lambda shape: pl.BlockSpec(shape, lambda i: (0, 0))
    return pl.pallas_call(
        _mlp_body,
        grid=(_B // _BS,),
        in_specs=[
            pl.BlockSpec((_D, _BS), lambda i: (0, i)),
            pl.BlockSpec((_D, _BS), lambda i: (0, i)),
            full(w1at.shape), full(w1bt.shape), full(b1c.shape),
            full(w2t.shape), full(b2c.shape),
            full(w3t.shape), full(b3c.shape),
            full(w4t.shape), full(b4c.shape),
        ],
        out_specs=pl.BlockSpec((1, _BS), lambda i: (0, i)),
        out_shape=jax.ShapeDtypeStruct((1, _B), jnp.float32),
    )(ut, gt, w1at, w1bt, b1c, w2t, b2c, w3t, b3c, w4t, b4c)


def kernel(user, game, player_table, game_table, W1, b1, W2, b2, W3, b3, W4, b4):
    uidx = user.reshape(_B)
    gidx = game.reshape(_B)
    ptab3 = player_table.reshape(player_table.shape[0] // 8, 8, _D)
    gtab3 = game_table.reshape(game_table.shape[0] // 8, 8, _D)
    ut, gt = _make_sc_gather()(uidx, gidx, ptab3, gtab3)
    out_t = _mlp_t(ut, gt,
                   W1[:_D].T, W1[_D:].T, b1.reshape(-1, 1),
                   W2.T, b2.reshape(-1, 1),
                   W3.T, b3.reshape(-1, 1),
                   W4.T, b4.reshape(1, 1))
    return out_t.reshape(_B, 1)
